# baseline (speedup 1.0000x reference)
import functools

import jax
import jax.numpy as jnp
from jax import lax
from jax.experimental import pallas as pl
from jax.experimental.pallas import tpu as pltpu
from jax.experimental.pallas import tpu_sc as plsc

_BATCH = 16384
_HIST = 200
_EMBED = 32
_B = _BATCH * _HIST
_LANES = 128
_SUB = 8
_CHUNK = _SUB * _LANES
_NW = 32
_NCHUNKS = _B // _CHUNK
_CPW = _NCHUNKS // _NW
_HG = _HIST // _SUB
_EG = _EMBED // 8


_TC = 512
_TCPW = 61
_TREM = 576
_TROFF = _NW * _TCPW * _TC


def _tpose_body(tabt_hbm, trm_hbm, in_v, outb_v, sem_in0, sem_in1,
                sem_out0, sem_out1):
    nc = plsc.get_sparse_core_info().num_cores
    wid = lax.axis_index("s") * nc + lax.axis_index("c")
    sem_in = (sem_in0, sem_in1)
    sem_out = (sem_out0, sem_out1)
    iota = lax.iota(jnp.int32, 16)

    def c0_of(k):
        return (wid + _NW * k) * _TC

    def start_in(c0, s, w):
        pltpu.async_copy(
            tabt_hbm.at[pl.ds(0, _EMBED), pl.ds(c0, w)],
            in_v.at[s, pl.ds(0, _EMBED), pl.ds(0, w)], sem_in[s])

    def wait_in(s, w):
        pltpu.make_async_copy(
            tabt_hbm.at[pl.ds(0, _EMBED), pl.ds(0, w)],
            in_v.at[s, pl.ds(0, _EMBED), pl.ds(0, w)], sem_in[s]).wait()

    def wait_out(s, w):
        pltpu.make_async_copy(
            outb_v.at[s, pl.ds(0, w), pl.ds(0, _EMBED)],
            trm_hbm.at[pl.ds(0, w)], sem_out[s]).wait()

    def transpose_block(s, w):
        for e in range(_EMBED):
            fe = jnp.full((16,), e, jnp.int32)

            @plsc.parallel_loop(0, w // 16, 1, unroll=8)
            def _(g):
                cg = iota + g * 16
                v = in_v[s, e, pl.ds(g * 16, 16)]
                plsc.store_scatter(outb_v.at[s], [cg, fe], v)

    def step(k, s):
        wait_in(s, _TC)

        @pl.when(k >= 2)
        def _():
            wait_out(s, _TC)

        transpose_block(s, _TC)
        pltpu.async_copy(
            outb_v.at[s, pl.ds(0, _TC), pl.ds(0, _EMBED)],
            trm_hbm.at[pl.ds(c0_of(k), _TC)], sem_out[s])

        @pl.when(k + 2 < _TCPW)
        def _():
            start_in(c0_of(k + 2), s, _TC)

    start_in(c0_of(0), 0, _TC)
    start_in(c0_of(1), 1, _TC)
    step(0, 0)

    def loop_body(i, carry):
        step(2 * i + 1, 1)
        step(2 * i + 2, 0)
        return carry

    lax.fori_loop(0, _TCPW // 2, loop_body, 0)
    wait_out(0, _TC)
    wait_out(1, _TC)

    @pl.when(wid == 0)
    def _():
        start_in(_TROFF, 0, _TREM)
        wait_in(0, _TREM)
        transpose_block(0, _TREM)
        pltpu.async_copy(
            outb_v.at[0, pl.ds(0, _TREM), pl.ds(0, _EMBED)],
            trm_hbm.at[pl.ds(_TROFF, _TREM)], sem_out[0])
        wait_out(0, _TREM)


def _body(idx_hbm, table_hbm, out_hbm, idx_v, rows_v, tp_v,
          sem_i0, sem_i1, sem_g0, sem_g1, sem_t):
    nc = plsc.get_sparse_core_info().num_cores
    wid = lax.axis_index("s") * nc + lax.axis_index("c")
    sem_i = (sem_i0, sem_i1)
    sem_g = (sem_g0, sem_g1)
    iota = lax.iota(jnp.int32, 16)
    half_e = [iota + h * 16 for h in range(2)]

    def chunk_id(ch):
        return wid * _CPW + ch

    def start_idx(ch, slot):
        pltpu.async_copy(
            idx_hbm.at[pl.ds(chunk_id(ch) * _SUB, _SUB)], idx_v.at[slot],
            sem_i[slot])

    def wait_idx(slot):
        pltpu.make_async_copy(
            idx_hbm.at[pl.ds(0, _SUB)], idx_v.at[slot], sem_i[slot]).wait()

    def fire(slot):
        for j in range(_SUB):
            pltpu.async_copy(
                table_hbm.at[idx_v.at[slot, j]],
                rows_v.at[slot, pl.ds(j * _LANES, _LANES)],
                sem_g[slot])

    def drain_gathers(slot):
        pltpu.make_async_copy(
            table_hbm.at[pl.ds(0, _CHUNK)], rows_v.at[slot],
            sem_g[slot]).wait()

    def drain_tp():
        for _ in range(_SUB * _EG):
            pltpu.make_async_copy(
                tp_v.at[0, pl.ds(0, _SUB), pl.ds(0, _LANES)],
                out_hbm.at[0, 0, 0], sem_t).wait()

    def transpose_store(ch, slot, g):
        cid = chunk_id(ch)
        hg = cid // _LANES
        bt = cid - hg * _LANES

        @pl.when(g >= 1)
        def _():
            drain_tp()

        for h8 in range(_SUB):

            @plsc.parallel_loop(0, _LANES, 1, unroll=8)
            def _(b):
                bs = jnp.full((16,), 0, jnp.int32) + b
                for hf in range(2):
                    v = rows_v[slot, h8 * _LANES + b, pl.ds(hf * 16, 16)]
                    plsc.store_scatter(tp_v.at[h8], [half_e[hf], bs], v)
            h = hg * _SUB + h8
            for eg in range(_EG):
                pltpu.async_copy(
                    tp_v.at[h8, pl.ds(eg * _SUB, _SUB), pl.ds(0, _LANES)],
                    out_hbm.at[h, eg, bt], sem_t)

    def step(g, slot):
        @pl.when(g + 1 < _CPW)
        def _():
            wait_idx(slot ^ 1)
            fire(slot ^ 1)

        drain_gathers(slot)

        @pl.when(g + 2 < _CPW)
        def _():
            start_idx(g + 2, slot)

        transpose_store(g, slot, g)

    start_idx(0, 0)
    start_idx(1, 1)
    wait_idx(0)
    fire(0)

    def loop_body(i, carry):
        step(2 * i, 0)
        step(2 * i + 1, 1)
        return carry

    lax.fori_loop(0, _CPW // 2, loop_body, 0)
    drain_tp()


@functools.partial(jax.jit, static_argnames=())
def kernel(action, table):
    act_view = (action.astype(jnp.int32)
                .reshape(_LANES, _LANES, _HG, _SUB)
                .transpose(2, 0, 3, 1)
                .reshape(_B // _LANES, _LANES))
    mesh = plsc.VectorSubcoreMesh(core_axis_name="c", subcore_axis_name="s")
    table_rm = pl.kernel(
        _tpose_body,
        out_type=jax.ShapeDtypeStruct((NUM := 1000000, _EMBED), jnp.float32),
        mesh=mesh,
        scratch_types=[
            pltpu.VMEM((2, _EMBED, _TREM), jnp.float32),
            pltpu.VMEM((2, _TREM, _EMBED + 1), jnp.float32),
            pltpu.SemaphoreType.DMA,
            pltpu.SemaphoreType.DMA,
            pltpu.SemaphoreType.DMA,
            pltpu.SemaphoreType.DMA,
        ],
        compiler_params=pltpu.CompilerParams(use_tc_tiling_on_sc=False,
                                             needs_layout_passes=False),
    )(jnp.transpose(table))
    out4 = pl.kernel(
        _body,
        out_type=jax.ShapeDtypeStruct((_HIST, _EG, _LANES, _SUB, _LANES),
                                      jnp.float32),
        mesh=mesh,
        scratch_types=[
            pltpu.VMEM((2, _SUB, _LANES), jnp.int32),
            pltpu.VMEM((2, _CHUNK, _EMBED), jnp.float32),
            pltpu.VMEM((_SUB, _EMBED, _LANES + 1), jnp.float32),
            pltpu.SemaphoreType.DMA,
            pltpu.SemaphoreType.DMA,
            pltpu.SemaphoreType.DMA,
            pltpu.SemaphoreType.DMA,
            pltpu.SemaphoreType.DMA,
        ],
        compiler_params=pltpu.CompilerParams(use_tc_tiling_on_sc=False,
                                             needs_layout_passes=False),
    )(act_view, table_rm)
    return (out4.transpose(2, 4, 0, 1, 3)
            .reshape(_BATCH, _HIST, _EMBED))


# speedup vs baseline: 4.0180x; 4.0180x over previous
import functools

import jax
import jax.numpy as jnp
from jax import lax
from jax.experimental import pallas as pl
from jax.experimental.pallas import tpu as pltpu
from jax.experimental.pallas import tpu_sc as plsc

_BATCH = 16384
_HIST = 200
_EMBED = 32
_B = _BATCH * _HIST
_LANES = 128
_SUB = 8
_CHUNK = _SUB * _LANES
_NW = 32
_NCHUNKS = _B // _CHUNK
_CPW = _NCHUNKS // _NW
_HG = _HIST // _SUB
_EG = _EMBED // 8


def _body(idx_hbm, table_hbm, out_hbm, idx_v, rows_v, tp_v,
          sem_i0, sem_i1, sem_g0, sem_g1, sem_t):
    nc = plsc.get_sparse_core_info().num_cores
    wid = lax.axis_index("s") * nc + lax.axis_index("c")
    sem_i = (sem_i0, sem_i1)
    sem_g = (sem_g0, sem_g1)
    iota = lax.iota(jnp.int32, 16)
    half_e = [iota + h * 16 for h in range(2)]

    def chunk_id(ch):
        return wid * _CPW + ch

    def start_idx(ch, slot):
        pltpu.async_copy(
            idx_hbm.at[pl.ds(chunk_id(ch) * _SUB, _SUB)], idx_v.at[slot],
            sem_i[slot])

    def wait_idx(slot):
        pltpu.make_async_copy(
            idx_hbm.at[pl.ds(0, _SUB)], idx_v.at[slot], sem_i[slot]).wait()

    def fire(slot):
        for j in range(_SUB):
            pltpu.async_copy(
                table_hbm.at[idx_v.at[slot, j]],
                rows_v.at[slot, pl.ds(j * _LANES, _LANES)],
                sem_g[slot])

    def drain_gathers(slot):
        pltpu.make_async_copy(
            table_hbm.at[pl.ds(0, _CHUNK)], rows_v.at[slot],
            sem_g[slot]).wait()

    def drain_tp():
        for _ in range(_SUB * _EG):
            pltpu.make_async_copy(
                tp_v.at[0, pl.ds(0, _SUB), pl.ds(0, _LANES)],
                out_hbm.at[0, 0, 0], sem_t).wait()

    def transpose_store(ch, slot, g):
        cid = chunk_id(ch)
        hg = cid // _LANES
        bt = cid - hg * _LANES

        @pl.when(g >= 1)
        def _():
            drain_tp()

        for h8 in range(_SUB):

            @plsc.parallel_loop(0, _LANES, 1, unroll=8)
            def _(b):
                bs = jnp.full((16,), 0, jnp.int32) + b
                for hf in range(2):
                    v = rows_v[slot, h8 * _LANES + b, pl.ds(hf * 16, 16)]
                    plsc.store_scatter(tp_v.at[h8], [half_e[hf], bs], v)
            h = hg * _SUB + h8
            for eg in range(_EG):
                pltpu.async_copy(
                    tp_v.at[h8, pl.ds(eg * _SUB, _SUB), pl.ds(0, _LANES)],
                    out_hbm.at[h, eg, bt], sem_t)

    def scale_idx(slot):
        for j in range(_SUB):

            @plsc.parallel_loop(0, _LANES // 16, 1, unroll=4)
            def _(g):
                v = idx_v[slot, j, pl.ds(g * 16, 16)]
                idx_v[slot, j, pl.ds(g * 16, 16)] = v * 4

    def step(g, slot):
        @pl.when(g + 1 < _CPW)
        def _():
            wait_idx(slot ^ 1)
            scale_idx(slot ^ 1)
            fire(slot ^ 1)

        drain_gathers(slot)

        @pl.when(g + 2 < _CPW)
        def _():
            start_idx(g + 2, slot)

        transpose_store(g, slot, g)

    start_idx(0, 0)
    start_idx(1, 1)
    wait_idx(0)
    scale_idx(0)
    fire(0)

    def loop_body(i, carry):
        step(2 * i, 0)
        step(2 * i + 1, 1)
        return carry

    lax.fori_loop(0, _CPW // 2, loop_body, 0)
    drain_tp()


@functools.partial(jax.jit, static_argnames=())
def kernel(action, table):
    act_view = (action.astype(jnp.int32)
                .reshape(_LANES, _LANES, _HG, _SUB)
                .transpose(2, 0, 3, 1)
                .reshape(_B // _LANES, _LANES))
    mesh = plsc.VectorSubcoreMesh(core_axis_name="c", subcore_axis_name="s")
    out4 = pl.kernel(
        _body,
        out_type=jax.ShapeDtypeStruct((_HIST, _EG, _LANES, _SUB, _LANES),
                                      jnp.float32),
        mesh=mesh,
        scratch_types=[
            pltpu.VMEM((2, _SUB, _LANES), jnp.int32),
            pltpu.VMEM((2, _CHUNK, _EMBED), jnp.float32),
            pltpu.VMEM((_SUB, _EMBED, _LANES + 1), jnp.float32),
            pltpu.SemaphoreType.DMA,
            pltpu.SemaphoreType.DMA,
            pltpu.SemaphoreType.DMA,
            pltpu.SemaphoreType.DMA,
            pltpu.SemaphoreType.DMA,
        ],
        compiler_params=pltpu.CompilerParams(use_tc_tiling_on_sc=False,
                                             needs_layout_passes=False),
    )(act_view,
      jnp.pad(table, ((0, 0), (0, 96))).reshape(4 * 1000000, _EMBED))
    return (out4.transpose(2, 4, 0, 1, 3)
            .reshape(_BATCH, _HIST, _EMBED))


# speedup vs baseline: 4.3513x; 1.0830x over previous
import functools

import jax
import jax.numpy as jnp
from jax import lax
from jax.experimental import pallas as pl
from jax.experimental.pallas import tpu as pltpu
from jax.experimental.pallas import tpu_sc as plsc

_BATCH = 16384
_HIST = 200
_EMBED = 32
_B = _BATCH * _HIST
_LANES = 128
_SUB = 8
_CHUNK = _SUB * _LANES
_NW = 32
_NCHUNKS = _B // _CHUNK
_CPW = _NCHUNKS // _NW
_HG = _HIST // _SUB
_EG = _EMBED // 8


_CT = 4
_TCHUNKS = 1952
_TCPW = _TCHUNKS // _NW
_NTILES = 7813


def _tpose_body(tabn_hbm, trm_hbm, in_v, outb_v, sem_in0, sem_in1,
                sem_out0, sem_out1):
    nc = plsc.get_sparse_core_info().num_cores
    wid = lax.axis_index("s") * nc + lax.axis_index("c")
    sem_in = (sem_in0, sem_in1)
    sem_out = (sem_out0, sem_out1)
    iota = lax.iota(jnp.int32, 16)

    def start_in(ct0, s, w):
        pltpu.async_copy(
            tabn_hbm.at[pl.ds(0, _EG), pl.ds(ct0, w), pl.ds(0, 8),
                        pl.ds(0, _LANES)],
            in_v.at[s, pl.ds(0, _EG), pl.ds(0, w)], sem_in[s])

    def wait_in(s, w):
        pltpu.make_async_copy(
            tabn_hbm.at[pl.ds(0, _EG), pl.ds(0, w), pl.ds(0, 8),
                        pl.ds(0, _LANES)],
            in_v.at[s, pl.ds(0, _EG), pl.ds(0, w)], sem_in[s]).wait()

    def wait_out(s, w):
        pltpu.make_async_copy(
            outb_v.at[s, pl.ds(0, w * _LANES), pl.ds(0, _EMBED)],
            trm_hbm.at[pl.ds(0, w * _LANES)], sem_out[s]).wait()

    def transpose_block(s, w):
        for eg in range(_EG):
            for e8 in range(8):
                fe = jnp.full((16,), eg * 8 + e8, jnp.int32)

                @plsc.parallel_loop(0, w * _SUB, 1, unroll=8)
                def _(i):
                    ct = i // _SUB
                    g = i - ct * _SUB
                    cg = iota + (ct * _LANES + g * 16)
                    v = in_v[s, eg, ct, e8, pl.ds(g * 16, 16)]
                    plsc.store_scatter(outb_v.at[s], [cg, fe], v)

    def step(k, s):
        ct0 = wid * _TCPW * _CT + k * _CT
        wait_in(s, _CT)

        @pl.when(k >= 2)
        def _():
            wait_out(s, _CT)

        transpose_block(s, _CT)
        pltpu.async_copy(
            outb_v.at[s, pl.ds(0, _CT * _LANES), pl.ds(0, _EMBED)],
            trm_hbm.at[pl.ds(ct0 * _LANES, _CT * _LANES)], sem_out[s])

        @pl.when(k + 2 < _TCPW)
        def _():
            start_in(wid * _TCPW * _CT + (k + 2) * _CT, s, _CT)

    start_in(wid * _TCPW * _CT, 0, _CT)
    start_in(wid * _TCPW * _CT + _CT, 1, _CT)
    step(0, 0)

    def loop_body(i, carry):
        step(2 * i + 1, 1)
        step(2 * i + 2, 0)
        return carry

    lax.fori_loop(0, _TCPW // 2, loop_body, 0)
    wait_out(0, _CT)
    wait_out(1, _CT)

    @pl.when(wid == 0)
    def _():
        start_in(_TCHUNKS * _CT, 0, 5)
        wait_in(0, 5)
        transpose_block(0, 5)
        pltpu.async_copy(
            outb_v.at[0, pl.ds(0, 576), pl.ds(0, _EMBED)],
            trm_hbm.at[pl.ds(_TCHUNKS * _CT * _LANES, 576)], sem_out[0])
        pltpu.make_async_copy(
            outb_v.at[0, pl.ds(0, 576), pl.ds(0, _EMBED)],
            trm_hbm.at[pl.ds(0, 576)], sem_out[0]).wait()


def _body(idx_hbm, table_hbm, out_hbm, idx_v, rows_v, tp_v,
          sem_i0, sem_i1, sem_g0, sem_g1, sem_t):
    nc = plsc.get_sparse_core_info().num_cores
    wid = lax.axis_index("s") * nc + lax.axis_index("c")
    sem_i = (sem_i0, sem_i1)
    sem_g = (sem_g0, sem_g1)
    iota = lax.iota(jnp.int32, 16)
    half_e = [iota + h * 16 for h in range(2)]

    def chunk_id(ch):
        return wid * _CPW + ch

    def start_idx(ch, slot):
        pltpu.async_copy(
            idx_hbm.at[pl.ds(chunk_id(ch) * _SUB, _SUB)], idx_v.at[slot],
            sem_i[slot])

    def wait_idx(slot):
        pltpu.make_async_copy(
            idx_hbm.at[pl.ds(0, _SUB)], idx_v.at[slot], sem_i[slot]).wait()

    def fire(slot):
        for j in range(_SUB):
            pltpu.async_copy(
                table_hbm.at[idx_v.at[slot, j]],
                rows_v.at[slot, pl.ds(j * _LANES, _LANES)],
                sem_g[slot])

    def drain_gathers(slot):
        pltpu.make_async_copy(
            table_hbm.at[pl.ds(0, _CHUNK)], rows_v.at[slot],
            sem_g[slot]).wait()

    def drain_tp():
        for _ in range(_SUB * _EG):
            pltpu.make_async_copy(
                tp_v.at[0, pl.ds(0, _SUB), pl.ds(0, _LANES)],
                out_hbm.at[0, 0, 0], sem_t).wait()

    def transpose_store(ch, slot, g):
        cid = chunk_id(ch)
        hg = cid // _LANES
        bt = cid - hg * _LANES

        @pl.when(g >= 1)
        def _():
            drain_tp()

        for h8 in range(_SUB):

            @plsc.parallel_loop(0, _LANES, 1, unroll=8)
            def _(b):
                bs = jnp.full((16,), 0, jnp.int32) + b
                for hf in range(2):
                    v = rows_v[slot, h8 * _LANES + b, pl.ds(hf * 16, 16)]
                    plsc.store_scatter(tp_v.at[h8], [half_e[hf], bs], v)
            h = hg * _SUB + h8
            for eg in range(_EG):
                pltpu.async_copy(
                    tp_v.at[h8, pl.ds(eg * _SUB, _SUB), pl.ds(0, _LANES)],
                    out_hbm.at[h, eg, bt], sem_t)

    def step(g, slot):
        @pl.when(g + 1 < _CPW)
        def _():
            wait_idx(slot ^ 1)
            fire(slot ^ 1)

        drain_gathers(slot)

        @pl.when(g + 2 < _CPW)
        def _():
            start_idx(g + 2, slot)

        transpose_store(g, slot, g)

    start_idx(0, 0)
    start_idx(1, 1)
    wait_idx(0)
    fire(0)

    def loop_body(i, carry):
        step(2 * i, 0)
        step(2 * i + 1, 1)
        return carry

    lax.fori_loop(0, _CPW // 2, loop_body, 0)
    drain_tp()


@functools.partial(jax.jit, static_argnames=())
def kernel(action, table):
    act_view = (action.astype(jnp.int32)
                .reshape(_LANES, _LANES, _HG, _SUB)
                .transpose(2, 0, 3, 1)
                .reshape(_B // _LANES, _LANES))
    mesh = plsc.VectorSubcoreMesh(core_axis_name="c", subcore_axis_name="s")
    tabn = (jnp.pad(jnp.transpose(table), ((0, 0), (0, 64)))
            .reshape(_EG, 8, _NTILES, _LANES)
            .transpose(0, 2, 1, 3))
    table_rm = pl.kernel(
        _tpose_body,
        out_type=jax.ShapeDtypeStruct((1000000, _EMBED), jnp.float32),
        mesh=mesh,
        scratch_types=[
            pltpu.VMEM((2, _EG, 5, 8, _LANES), jnp.float32),
            pltpu.VMEM((2, 640, _EMBED + 1), jnp.float32),
            pltpu.SemaphoreType.DMA,
            pltpu.SemaphoreType.DMA,
            pltpu.SemaphoreType.DMA,
            pltpu.SemaphoreType.DMA,
        ],
        compiler_params=pltpu.CompilerParams(use_tc_tiling_on_sc=False,
                                             needs_layout_passes=False),
    )(tabn)
    out4 = pl.kernel(
        _body,
        out_type=jax.ShapeDtypeStruct((_HIST, _EG, _LANES, _SUB, _LANES),
                                      jnp.float32),
        mesh=mesh,
        scratch_types=[
            pltpu.VMEM((2, _SUB, _LANES), jnp.int32),
            pltpu.VMEM((2, _CHUNK, _EMBED), jnp.float32),
            pltpu.VMEM((_SUB, _EMBED, _LANES + 1), jnp.float32),
            pltpu.SemaphoreType.DMA,
            pltpu.SemaphoreType.DMA,
            pltpu.SemaphoreType.DMA,
            pltpu.SemaphoreType.DMA,
            pltpu.SemaphoreType.DMA,
        ],
        compiler_params=pltpu.CompilerParams(use_tc_tiling_on_sc=False,
                                             needs_layout_passes=False),
    )(act_view, table_rm)
    return (out4.transpose(2, 4, 0, 1, 3)
            .reshape(_BATCH, _HIST, _EMBED))
